# baseline (speedup 1.0000x reference)
import functools

import numpy as np
import jax
import jax.numpy as jnp
from jax.experimental import pallas as pl
from jax.experimental.pallas import tpu as pltpu

_VMEM_BYTES = 60 * 1024 * 1024
_ROWS_PER_STEP = 128


def _se_mxu_kernel(x_ref, s_ref, st_ref, w1e_ref, m1_ref, f1_ref, ebt_ref,
                   e_ref, b1_ref, w2e_ref, m2_ref, f2_ref, b2t_ref, o_ref, *,
                   groups):
    dot = functools.partial(jax.lax.dot, precision=None,
                            preferred_element_type=jnp.float32)
    xb = x_ref[...]
    sums = dot(xb, s_ref[...])
    hyp1 = dot(sums, w1e_ref[...])
    hp = dot(hyp1 * m1_ref[...], f1_ref[...])
    h = dot(ebt_ref[...], hp)
    h = jnp.maximum(h + b1_ref[...], 0.0)
    he = dot(e_ref[...], h)
    hyp2 = dot(he, w2e_ref[...])
    g = dot(hyp2 * m2_ref[...], f2_ref[...])
    g = jax.nn.sigmoid(g + b2t_ref[...])
    e2 = dot(g, st_ref[...])
    o_ref[...] = xb * e2


def kernel(x, w1, b1, w2, b2):
    B, C, H, W = x.shape
    HW = H * W
    Cr = w1.shape[0]

    gw = 128 // int(np.gcd(HW, 128))
    groups = C // gw
    lw = gw * HW
    rows = B * groups
    rps = _ROWS_PER_STEP
    if rows % rps != 0 or rps % groups != 0:
        rps = groups
    bps = rps // groups

    x2 = x.reshape(rows, lw)

    l = np.arange(lw)
    seg = np.equal.outer(l // HW, np.arange(gw)).astype(np.float32)
    seg_t = np.ascontiguousarray(seg.T)
    srow = np.arange(rps) % groups
    m1 = np.equal.outer(srow, np.arange(groups * Cr) // Cr).astype(np.float32)
    f1 = np.equal.outer(np.arange(groups * Cr) % Cr, np.arange(Cr)).astype(np.float32)
    m2 = np.equal.outer(srow, np.arange(groups * gw) // gw).astype(np.float32)
    f2 = np.equal.outer(np.arange(groups * gw) % gw, np.arange(gw)).astype(np.float32)
    eb = np.equal.outer(np.arange(rps) // groups, np.arange(bps)).astype(np.float32)
    ebt = np.ascontiguousarray(eb.T)

    w1e = jnp.transpose(w1.reshape(Cr, groups, gw), (2, 1, 0)).reshape(gw, groups * Cr)
    w1e = w1e * jnp.float32(1.0 / HW)
    w2e = jnp.transpose(w2.reshape(groups, gw, Cr), (2, 0, 1)).reshape(Cr, groups * gw)
    b2t = jnp.tile(b2.reshape(groups, gw), (bps, 1))
    b1r = b1.reshape(1, Cr)

    const = lambda i: (0, 0)
    out = pl.pallas_call(
        functools.partial(_se_mxu_kernel, groups=groups),
        out_shape=jax.ShapeDtypeStruct((rows, lw), x.dtype),
        grid=(rows // rps,),
        in_specs=[
            pl.BlockSpec((rps, lw), lambda i: (i, 0)),
            pl.BlockSpec(seg.shape, const),
            pl.BlockSpec(seg_t.shape, const),
            pl.BlockSpec((gw, groups * Cr), const),
            pl.BlockSpec(m1.shape, const),
            pl.BlockSpec(f1.shape, const),
            pl.BlockSpec(ebt.shape, const),
            pl.BlockSpec(eb.shape, const),
            pl.BlockSpec((1, Cr), const),
            pl.BlockSpec((Cr, groups * gw), const),
            pl.BlockSpec(m2.shape, const),
            pl.BlockSpec(f2.shape, const),
            pl.BlockSpec((rps, gw), const),
        ],
        out_specs=pl.BlockSpec((rps, lw), lambda i: (i, 0)),
        compiler_params=pltpu.CompilerParams(
            dimension_semantics=("parallel",),
            vmem_limit_bytes=_VMEM_BYTES,
        ),
    )(x2, seg, seg_t, w1e, m1, f1, ebt, eb, b1r, w2e, m2, f2, b2t)
    return out.reshape(B, C, H, W)


# speedup vs baseline: 25.8817x; 25.8817x over previous
import functools

import jax
import jax.numpy as jnp
from jax.experimental import pallas as pl
from jax.experimental.pallas import tpu as pltpu

_VMEM_BYTES = 56 * 1024 * 1024
_BT = 16


def _se_native_kernel(x_ref, w1t_ref, b1_ref, w2t_ref, b2_ref, o_ref, *, inv_hw):
    dot = functools.partial(jax.lax.dot, preferred_element_type=jnp.float32)
    mean = jnp.sum(x_ref[...], axis=0) * inv_hw
    h = jnp.maximum(dot(mean, w1t_ref[...]) + b1_ref[...], 0.0)
    g = jax.nn.sigmoid(dot(h, w2t_ref[...]) + b2_ref[...])
    o_ref[...] = x_ref[...] * g[None]


def kernel(x, w1, b1, w2, b2):
    B, C, H, W = x.shape
    HW = H * W
    Cr = w1.shape[0]
    bt = _BT if B % _BT == 0 else B

    xt = jnp.transpose(x, (2, 3, 0, 1)).reshape(HW, B, C)
    w1t = jnp.transpose(w1)
    w2t = jnp.transpose(w2)

    const = lambda i: (0, 0)
    out = pl.pallas_call(
        functools.partial(_se_native_kernel, inv_hw=1.0 / HW),
        out_shape=jax.ShapeDtypeStruct((HW, B, C), x.dtype),
        grid=(B // bt,),
        in_specs=[
            pl.BlockSpec((HW, bt, C), lambda i: (0, i, 0)),
            pl.BlockSpec((C, Cr), const),
            pl.BlockSpec((1, Cr), const),
            pl.BlockSpec((Cr, C), const),
            pl.BlockSpec((1, C), const),
        ],
        out_specs=pl.BlockSpec((HW, bt, C), lambda i: (0, i, 0)),
        compiler_params=pltpu.CompilerParams(
            dimension_semantics=("parallel",),
            vmem_limit_bytes=_VMEM_BYTES,
        ),
    )(xt, w1t, b1.reshape(1, Cr), w2t, b2.reshape(1, C))
    return jnp.transpose(out.reshape(H, W, B, C), (2, 3, 0, 1))
